# baseline (speedup 1.0000x reference)
import functools

import jax
import jax.numpy as jnp
from jax import lax
from jax.experimental import pallas as pl
from jax.experimental.pallas import tpu as pltpu
from jax.experimental.pallas import tpu_sc as plsc

S = 8192
TOPK = 128
M = 160
L = 16
NW = 16
SLICE = S // NW
NCW = SLICE // L
NCH = S // L
NB_M = M // L
NB_K = TOPK // L
FLAT_PAD = 2**30
NEG_INF = float("-inf")


def _iota16():
    return lax.iota(jnp.int32, 16)


_GDN = lax.GatherDimensionNumbers(
    offset_dims=(), collapsed_slice_dims=(0,), start_index_map=(0,))


def _perm(x, idx):
    return lax.gather(x, idx[:, None], _GDN, (1,),
                      mode=lax.GatherScatterMode.PROMISE_IN_BOUNDS)


def _before(k1, v1, k2, v2):
    return (k1 > k2) | ((k1 == k2) & (v1 <= v2))


def _cmpx(kk, vv, dist, desc_mask):
    idx = _iota16() ^ dist
    pk = _perm(kk, idx)
    pv = _perm(vv, idx)
    first = (_iota16() & dist) == 0
    win = _before(kk, vv, pk, pv)
    keep = win == (first == desc_mask)
    return jnp.where(keep, kk, pk), jnp.where(keep, vv, pv)


def _bmerge16(kk, vv):
    for dist in (8, 4, 2, 1):
        kk, vv = _cmpx(kk, vv, dist, True)
    return kk, vv


def _sort16(kk, vv):
    io = _iota16()
    for blk in (2, 4, 8, 16):
        desc_mask = (io & blk) == 0
        dist = blk // 2
        while dist >= 1:
            kk, vv = _cmpx(kk, vv, dist, desc_mask)
            dist //= 2
    return kk, vv


def _merge2x16(ak, av, bk, bv):
    rbk = jnp.flip(bk, 0)
    rbv = jnp.flip(bv, 0)
    take = _before(ak, av, rbk, rbv)
    hk = jnp.where(take, ak, rbk)
    hv = jnp.where(take, av, rbv)
    lk = jnp.where(take, rbk, ak)
    lv = jnp.where(take, rbv, av)
    hk, hv = _bmerge16(hk, hv)
    lk, lv = _bmerge16(lk, lv)
    return hk, hv, lk, lv


def _merge2x16_hw(ak, av, bk, bv):
    rbk = jnp.flip(bk, 0)
    rbv = jnp.flip(bv, 0)
    take = ak >= rbk
    hk = jnp.where(take, ak, rbk)
    hv = jnp.where(take, av, rbv)
    lk = jnp.where(take, rbk, ak)
    lv = jnp.where(take, rbv, av)
    hk, hv = plsc.sort_key_val(hk, hv, descending=True)
    lk, lv = plsc.sort_key_val(lk, lv, descending=True)
    return hk, hv, lk, lv


def _insert(bufs, ck, cv, hw=False):
    merge = _merge2x16_hw if hw else _merge2x16
    out = []
    for bk, bv in bufs:
        hk, hv, ck, cv = merge(bk, bv, ck, cv)
        out.append((hk, hv))
    return out


def _flatten(bufs):
    return tuple(x for kv in bufs for x in kv)


def _unflatten(flat):
    return [(flat[2 * i], flat[2 * i + 1]) for i in range(len(flat) // 2)]


def _init_bufs(nbuf):
    return _flatten([(jnp.full((L,), NEG_INF, jnp.float32),
                      jnp.full((L,), FLAT_PAD, jnp.int32))
                     for _ in range(nbuf)])


def _merge_lists_scan(keys_ref, vals_ref, nbuf, nchunks, hw):
    def body(i, carry):
        bufs = _unflatten(carry)
        ck = keys_ref[pl.ds(i * L, L)]
        cv = vals_ref[pl.ds(i * L, L)]
        tau = jnp.min(bufs[-1][0])
        cmax = jnp.max(ck)

        def ins(c):
            if hw:
                sk, sv = plsc.sort_key_val(ck, cv, descending=True)
            else:
                sk, sv = _sort16(ck, cv)
            return _flatten(_insert(_unflatten(c), sk, sv, hw=hw))

        return lax.cond(cmax >= tau, ins, lambda c: c, carry)

    return _unflatten(lax.fori_loop(0, nchunks, body, _init_bufs(nbuf)))


def _range_max(ref, nchunks_t):
    def body(i, acc):
        return jnp.maximum(acc, jnp.max(ref[pl.ds(i * L, L)]))

    return lax.fori_loop(0, nchunks_t, body, jnp.float32(NEG_INF))


def _lists_body(b_hbm, e_hbm, ck_hbm, ci_hbm, dk_hbm, di_hbm,
                bfull, efull, k_st, i_st):
    core = lax.axis_index("c")
    sub = lax.axis_index("s")

    @pl.when(core == 0)
    def _():
        w = sub
        base = w * SLICE
        io = _iota16()
        pltpu.sync_copy(b_hbm, bfull)
        pltpu.sync_copy(e_hbm, efull)

        pmcarry = _range_max(bfull, w * NCW)

        def smbody(i, acc):
            off = base + SLICE + i * L
            return jnp.maximum(acc, jnp.max(efull[pl.ds(off, L)]))

        smcarry = lax.fori_loop(0, (NW - 1 - w) * NCW, smbody,
                                jnp.float32(NEG_INF))

        def fwd(i, carry):
            pmax = carry[0]
            bufs = _unflatten(carry[1:])
            bx = bfull[pl.ds(base + i * L, L)]
            ey = efull[pl.ds(base + i * L, L)]
            pm = jnp.maximum(plsc.cummax(bx), pmax)
            ck = pm + ey
            tau = jnp.min(bufs[-1][0])
            cmax = jnp.max(ck)

            def ins(c):
                sk, sv = plsc.sort_key_val(ck, base + i * L + io,
                                           descending=True)
                return _flatten(_insert(_unflatten(c), sk, sv, hw=True))

            newbufs = lax.cond(cmax >= tau, ins, lambda c: c, carry[1:])
            return (jnp.max(pm),) + tuple(newbufs)

        cres = lax.fori_loop(0, NCW, fwd, (pmcarry,) + _init_bufs(NB_M))
        jbufs = _unflatten(cres[1:])

        def bwd(t, carry):
            i = NCW - 1 - t
            smax = carry[0]
            bufs = _unflatten(carry[1:])
            bx = bfull[pl.ds(base + i * L, L)]
            ey = efull[pl.ds(base + i * L, L)]
            sm = jnp.maximum(jnp.flip(plsc.cummax(jnp.flip(ey, 0)), 0), smax)
            dk = bx + sm
            tau = jnp.min(bufs[-1][0])
            cmax = jnp.max(dk)

            def ins(c):
                sk, sv = plsc.sort_key_val(dk, base + i * L + io,
                                           descending=True)
                return _flatten(_insert(_unflatten(c), sk, sv, hw=True))

            newbufs = lax.cond(cmax >= tau, ins, lambda c: c, carry[1:])
            return (jnp.max(sm),) + tuple(newbufs)

        dres = lax.fori_loop(0, NCW, bwd, (smcarry,) + _init_bufs(NB_M))
        ibufs = _unflatten(dres[1:])

        for t in range(NB_M):
            k_st[pl.ds(t * L, L)] = jbufs[t][0]
            i_st[pl.ds(t * L, L)] = jbufs[t][1]
        pltpu.sync_copy(k_st, ck_hbm.at[pl.ds(w * M, M)])
        pltpu.sync_copy(i_st, ci_hbm.at[pl.ds(w * M, M)])
        for t in range(NB_M):
            k_st[pl.ds(t * L, L)] = ibufs[t][0]
            i_st[pl.ds(t * L, L)] = ibufs[t][1]
        pltpu.sync_copy(k_st, dk_hbm.at[pl.ds(w * M, M)])
        pltpu.sync_copy(i_st, di_hbm.at[pl.ds(w * M, M)])


def _final_body(b_hbm, e_hbm, ck_hbm, ci_hbm, dk_hbm, di_hbm,
                oi_hbm, oj_hbm, ov_hbm,
                bfull, efull, mk_l, mv_l, ej_l, jj_l, bi_l, ii_l,
                oiv, ojv, ovv):
    core = lax.axis_index("c")
    sub = lax.axis_index("s")

    @pl.when((core == 0) & (sub == 0))
    def _():
        pltpu.sync_copy(b_hbm, bfull)
        pltpu.sync_copy(e_hbm, efull)

        def esum(i, carry):
            seb, see = carry
            return (seb + jnp.exp(bfull[pl.ds(i * L, L)]),
                    see + jnp.exp(efull[pl.ds(i * L, L)]))

        seb, see = lax.fori_loop(0, NCH, esum,
                                 (jnp.zeros((L,), jnp.float32),
                                  jnp.zeros((L,), jnp.float32)))
        denom = jnp.sum(seb) * jnp.sum(see)

        pltpu.sync_copy(ck_hbm, mk_l)
        pltpu.sync_copy(ci_hbm, mv_l)
        gj = _merge_lists_scan(mk_l, mv_l, NB_M, NW * NB_M, hw=True)
        for t in range(NB_M):
            ji = gj[t][1]
            jj_l[pl.ds(t * L, L)] = ji
            ej_l[pl.ds(t * L, L)] = plsc.load_gather(efull, [ji])
        pltpu.sync_copy(dk_hbm, mk_l)
        pltpu.sync_copy(di_hbm, mv_l)
        gi = _merge_lists_scan(mk_l, mv_l, NB_M, NW * NB_M, hw=True)
        for t in range(NB_M):
            ii = gi[t][1]
            ii_l[pl.ds(t * L, L)] = ii
            bi_l[pl.ds(t * L, L)] = plsc.load_gather(bfull, [ii])

        emax_c = NEG_INF
        for t in range(NB_M):
            emax_c = jnp.maximum(emax_c, jnp.max(ej_l[pl.ds(t * L, L)]))

        def frow(r, carry):
            bufs = _unflatten(carry)
            tau0 = jnp.min(bufs[-1][0])
            bvec = plsc.load_gather(bi_l, [jnp.full((L,), r, jnp.int32)])

            def do_row(carry):
                ivec = plsc.load_gather(ii_l, [jnp.full((L,), r, jnp.int32)])
                for t in range(NB_M):
                    ek = ej_l[pl.ds(t * L, L)]
                    jv = jj_l[pl.ds(t * L, L)]
                    key = jnp.where(jv >= ivec, bvec + ek, NEG_INF)
                    flat = ivec * S + jv
                    bufs = _unflatten(carry)
                    tau = jnp.min(bufs[-1][0])
                    cmax = jnp.max(key)

                    def ins(c, key=key, flat=flat):
                        sk, sv = _sort16(key, flat)
                        return _flatten(_insert(_unflatten(c), sk, sv))

                    carry = lax.cond(cmax >= tau, ins, lambda c: c, carry)
                return carry

            return lax.cond(jnp.max(bvec) + emax_c >= tau0, do_row,
                            lambda c: c, carry)

        fbufs = _unflatten(lax.fori_loop(0, M, frow, _init_bufs(NB_K)))

        for t in range(NB_K):
            fk, fv = fbufs[t]
            oiv[pl.ds(t * L, L)] = lax.shift_right_logical(fv, 13)
            ojv[pl.ds(t * L, L)] = fv & (S - 1)
            ovv[pl.ds(t * L, L)] = jnp.exp(fk) / denom
        pltpu.sync_copy(oiv, oi_hbm)
        pltpu.sync_copy(ojv, oj_hbm)
        pltpu.sync_copy(ovv, ov_hbm)


@jax.jit
def _sc_select(b, e):
    mesh = plsc.VectorSubcoreMesh(core_axis_name="c", subcore_axis_name="s")
    lists = functools.partial(
        pl.kernel,
        mesh=mesh,
        compiler_params=pltpu.CompilerParams(needs_layout_passes=False),
        out_type=[
            jax.ShapeDtypeStruct((NW * M,), jnp.float32),
            jax.ShapeDtypeStruct((NW * M,), jnp.int32),
            jax.ShapeDtypeStruct((NW * M,), jnp.float32),
            jax.ShapeDtypeStruct((NW * M,), jnp.int32),
        ],
        scratch_types=[
            pltpu.VMEM((S,), jnp.float32),
            pltpu.VMEM((S,), jnp.float32),
            pltpu.VMEM((M,), jnp.float32),
            pltpu.VMEM((M,), jnp.int32),
        ],
    )(_lists_body)
    ck, ci, dk, di = lists(b, e)

    final = functools.partial(
        pl.kernel,
        mesh=mesh,
        compiler_params=pltpu.CompilerParams(needs_layout_passes=False),
        out_type=[
            jax.ShapeDtypeStruct((TOPK,), jnp.int32),
            jax.ShapeDtypeStruct((TOPK,), jnp.int32),
            jax.ShapeDtypeStruct((TOPK,), jnp.float32),
        ],
        scratch_types=[
            pltpu.VMEM((S,), jnp.float32),
            pltpu.VMEM((S,), jnp.float32),
            pltpu.VMEM((NW * M,), jnp.float32),
            pltpu.VMEM((NW * M,), jnp.int32),
            pltpu.VMEM((M,), jnp.float32),
            pltpu.VMEM((M,), jnp.int32),
            pltpu.VMEM((M,), jnp.float32),
            pltpu.VMEM((M,), jnp.int32),
            pltpu.VMEM((TOPK,), jnp.int32),
            pltpu.VMEM((TOPK,), jnp.int32),
            pltpu.VMEM((TOPK,), jnp.float32),
        ],
    )(_final_body)
    return final(b, e, ck, ci, dk, di)


def _tc_matvec(G, W):
    def body(g_ref, w_ref, o_ref):
        o_ref[...] = jnp.dot(g_ref[...], w_ref[...],
                             preferred_element_type=jnp.float32)

    return pl.pallas_call(
        body,
        out_shape=jax.ShapeDtypeStruct((S, 2), jnp.float32),
    )(G, W)


def kernel(G_p, Wb, We, k):
    del k
    be = _tc_matvec(G_p, jnp.concatenate([Wb, We], axis=1))
    b = be[:, 0]
    e = be[:, 1]
    oi, oj, vals = _sc_select(b, e)
    return (jnp.concatenate([oi[:, None], oj[:, None]], axis=1), vals)


# speedup vs baseline: 1.0038x; 1.0038x over previous
import functools

import jax
import jax.numpy as jnp
from jax import lax
from jax.experimental import pallas as pl
from jax.experimental.pallas import tpu as pltpu
from jax.experimental.pallas import tpu_sc as plsc

S = 8192
TOPK = 128
M = 160
L = 16
NW = 16
SLICE = S // NW
NCW = SLICE // L
NCH = S // L
NB_M = M // L
NB_K = TOPK // L
FLAT_PAD = 2**30
NEG_INF = float("-inf")


def _iota16():
    return lax.iota(jnp.int32, 16)


_GDN = lax.GatherDimensionNumbers(
    offset_dims=(), collapsed_slice_dims=(0,), start_index_map=(0,))


def _perm(x, idx):
    return lax.gather(x, idx[:, None], _GDN, (1,),
                      mode=lax.GatherScatterMode.PROMISE_IN_BOUNDS)


def _before(k1, v1, k2, v2):
    return (k1 > k2) | ((k1 == k2) & (v1 <= v2))


def _cmpx(kk, vv, dist, desc_mask):
    idx = _iota16() ^ dist
    pk = _perm(kk, idx)
    pv = _perm(vv, idx)
    first = (_iota16() & dist) == 0
    win = _before(kk, vv, pk, pv)
    keep = win == (first == desc_mask)
    return jnp.where(keep, kk, pk), jnp.where(keep, vv, pv)


def _bmerge16(kk, vv):
    for dist in (8, 4, 2, 1):
        kk, vv = _cmpx(kk, vv, dist, True)
    return kk, vv


def _sort16(kk, vv):
    io = _iota16()
    for blk in (2, 4, 8, 16):
        desc_mask = (io & blk) == 0
        dist = blk // 2
        while dist >= 1:
            kk, vv = _cmpx(kk, vv, dist, desc_mask)
            dist //= 2
    return kk, vv


def _merge2x16(ak, av, bk, bv):
    rbk = jnp.flip(bk, 0)
    rbv = jnp.flip(bv, 0)
    take = _before(ak, av, rbk, rbv)
    hk = jnp.where(take, ak, rbk)
    hv = jnp.where(take, av, rbv)
    lk = jnp.where(take, rbk, ak)
    lv = jnp.where(take, rbv, av)
    hk, hv = _bmerge16(hk, hv)
    lk, lv = _bmerge16(lk, lv)
    return hk, hv, lk, lv


def _merge2x16_hw(ak, av, bk, bv):
    rbk = jnp.flip(bk, 0)
    rbv = jnp.flip(bv, 0)
    take = ak >= rbk
    hk = jnp.where(take, ak, rbk)
    hv = jnp.where(take, av, rbv)
    lk = jnp.where(take, rbk, ak)
    lv = jnp.where(take, rbv, av)
    hk, hv = plsc.sort_key_val(hk, hv, descending=True)
    lk, lv = plsc.sort_key_val(lk, lv, descending=True)
    return hk, hv, lk, lv


def _insert(bufs, ck, cv, hw=False):
    merge = _merge2x16_hw if hw else _merge2x16
    out = []
    for bk, bv in bufs:
        hk, hv, ck, cv = merge(bk, bv, ck, cv)
        out.append((hk, hv))
    return out


def _flatten(bufs):
    return tuple(x for kv in bufs for x in kv)


def _unflatten(flat):
    return [(flat[2 * i], flat[2 * i + 1]) for i in range(len(flat) // 2)]


def _init_bufs(nbuf):
    return _flatten([(jnp.full((L,), NEG_INF, jnp.float32),
                      jnp.full((L,), FLAT_PAD, jnp.int32))
                     for _ in range(nbuf)])


def _merge_lists_scan(keys_ref, vals_ref, nbuf, nchunks, hw):
    def body(i, carry):
        bufs = _unflatten(carry)
        ck = keys_ref[pl.ds(i * L, L)]
        cv = vals_ref[pl.ds(i * L, L)]
        tau = jnp.min(bufs[-1][0])
        cmax = jnp.max(ck)

        def ins(c):
            if hw:
                sk, sv = plsc.sort_key_val(ck, cv, descending=True)
            else:
                sk, sv = _sort16(ck, cv)
            return _flatten(_insert(_unflatten(c), sk, sv, hw=hw))

        return lax.cond(cmax >= tau, ins, lambda c: c, carry)

    return _unflatten(lax.fori_loop(0, nchunks, body, _init_bufs(nbuf)))


def _range_max(ref, nchunks_t):
    def body(i, acc):
        return jnp.maximum(acc, jnp.max(ref[pl.ds(i * L, L)]))

    return lax.fori_loop(0, nchunks_t, body, jnp.float32(NEG_INF))


def _lists_body(b_hbm, e_hbm, ck_hbm, ci_hbm, dk_hbm, di_hbm,
                sb_hbm, se_hbm, bfull, efull, k_st, i_st, es_st):
    core = lax.axis_index("c")
    sub = lax.axis_index("s")

    @pl.when(core == 0)
    def _():
        w = sub
        base = w * SLICE
        io = _iota16()
        pltpu.sync_copy(b_hbm, bfull)
        pltpu.sync_copy(e_hbm, efull)

        pmcarry = _range_max(bfull, w * NCW)

        def smbody(i, acc):
            off = base + SLICE + i * L
            return jnp.maximum(acc, jnp.max(efull[pl.ds(off, L)]))

        smcarry = lax.fori_loop(0, (NW - 1 - w) * NCW, smbody,
                                jnp.float32(NEG_INF))

        def fwd(i, carry):
            pmax = carry[0]
            bufs = _unflatten(carry[1:])
            bx = bfull[pl.ds(base + i * L, L)]
            ey = efull[pl.ds(base + i * L, L)]
            pm = jnp.maximum(plsc.cummax(bx), pmax)
            ck = pm + ey
            tau = jnp.min(bufs[-1][0])
            cmax = jnp.max(ck)

            def ins(c):
                sk, sv = plsc.sort_key_val(ck, base + i * L + io,
                                           descending=True)
                return _flatten(_insert(_unflatten(c), sk, sv, hw=True))

            newbufs = lax.cond(cmax >= tau, ins, lambda c: c, carry[1:])
            return (jnp.max(pm),) + tuple(newbufs)

        cres = lax.fori_loop(0, NCW, fwd, (pmcarry,) + _init_bufs(NB_M))
        jbufs = _unflatten(cres[1:])

        def bwd(t, carry):
            i = NCW - 1 - t
            smax = carry[0]
            bufs = _unflatten(carry[1:])
            bx = bfull[pl.ds(base + i * L, L)]
            ey = efull[pl.ds(base + i * L, L)]
            sm = jnp.maximum(jnp.flip(plsc.cummax(jnp.flip(ey, 0)), 0), smax)
            dk = bx + sm
            tau = jnp.min(bufs[-1][0])
            cmax = jnp.max(dk)

            def ins(c):
                sk, sv = plsc.sort_key_val(dk, base + i * L + io,
                                           descending=True)
                return _flatten(_insert(_unflatten(c), sk, sv, hw=True))

            newbufs = lax.cond(cmax >= tau, ins, lambda c: c, carry[1:])
            return (jnp.max(sm),) + tuple(newbufs)

        dres = lax.fori_loop(0, NCW, bwd, (smcarry,) + _init_bufs(NB_M))
        ibufs = _unflatten(dres[1:])

        for t in range(NB_M):
            k_st[pl.ds(t * L, L)] = jbufs[t][0]
            i_st[pl.ds(t * L, L)] = jbufs[t][1]
        pltpu.sync_copy(k_st, ck_hbm.at[pl.ds(w * M, M)])
        pltpu.sync_copy(i_st, ci_hbm.at[pl.ds(w * M, M)])
        for t in range(NB_M):
            k_st[pl.ds(t * L, L)] = ibufs[t][0]
            i_st[pl.ds(t * L, L)] = ibufs[t][1]
        pltpu.sync_copy(k_st, dk_hbm.at[pl.ds(w * M, M)])
        pltpu.sync_copy(i_st, di_hbm.at[pl.ds(w * M, M)])

        def esum(i, carry):
            seb, see = carry
            return (seb + jnp.exp(bfull[pl.ds(base + i * L, L)]),
                    see + jnp.exp(efull[pl.ds(base + i * L, L)]))

        seb, see = lax.fori_loop(0, NCW, esum,
                                 (jnp.zeros((L,), jnp.float32),
                                  jnp.zeros((L,), jnp.float32)))
        es_st[...] = seb
        pltpu.sync_copy(es_st, sb_hbm.at[pl.ds(w * L, L)])
        es_st[...] = see
        pltpu.sync_copy(es_st, se_hbm.at[pl.ds(w * L, L)])


def _final_body(b_hbm, e_hbm, ck_hbm, ci_hbm, dk_hbm, di_hbm,
                sb_hbm, se_hbm, oi_hbm, oj_hbm, ov_hbm,
                bfull, efull, mk_l, mv_l, ej_l, jj_l, bi_l, ii_l,
                es_l, oiv, ojv, ovv):
    core = lax.axis_index("c")
    sub = lax.axis_index("s")

    @pl.when((core == 0) & (sub == 0))
    def _():
        pltpu.sync_copy(b_hbm, bfull)
        pltpu.sync_copy(e_hbm, efull)

        pltpu.sync_copy(sb_hbm, es_l)
        seb = jnp.zeros((L,), jnp.float32)
        for t in range(NW):
            seb = seb + es_l[pl.ds(t * L, L)]
        pltpu.sync_copy(se_hbm, es_l)
        see = jnp.zeros((L,), jnp.float32)
        for t in range(NW):
            see = see + es_l[pl.ds(t * L, L)]
        denom = jnp.sum(seb) * jnp.sum(see)

        pltpu.sync_copy(ck_hbm, mk_l)
        pltpu.sync_copy(ci_hbm, mv_l)
        gj = _merge_lists_scan(mk_l, mv_l, NB_M, NW * NB_M, hw=True)
        for t in range(NB_M):
            ji = gj[t][1]
            jj_l[pl.ds(t * L, L)] = ji
            ej_l[pl.ds(t * L, L)] = plsc.load_gather(efull, [ji])
        pltpu.sync_copy(dk_hbm, mk_l)
        pltpu.sync_copy(di_hbm, mv_l)
        gi = _merge_lists_scan(mk_l, mv_l, NB_M, NW * NB_M, hw=True)
        for t in range(NB_M):
            ii = gi[t][1]
            ii_l[pl.ds(t * L, L)] = ii
            bi_l[pl.ds(t * L, L)] = plsc.load_gather(bfull, [ii])

        emax_c = NEG_INF
        for t in range(NB_M):
            emax_c = jnp.maximum(emax_c, jnp.max(ej_l[pl.ds(t * L, L)]))

        def frow(r, carry):
            bufs = _unflatten(carry)
            tau0 = jnp.min(bufs[-1][0])
            bvec = plsc.load_gather(bi_l, [jnp.full((L,), r, jnp.int32)])

            def do_row(carry):
                ivec = plsc.load_gather(ii_l, [jnp.full((L,), r, jnp.int32)])
                for t in range(NB_M):
                    ek = ej_l[pl.ds(t * L, L)]
                    jv = jj_l[pl.ds(t * L, L)]
                    key = jnp.where(jv >= ivec, bvec + ek, NEG_INF)
                    flat = ivec * S + jv
                    bufs = _unflatten(carry)
                    tau = jnp.min(bufs[-1][0])
                    cmax = jnp.max(key)

                    def ins(c, key=key, flat=flat):
                        sk, sv = _sort16(key, flat)
                        return _flatten(_insert(_unflatten(c), sk, sv))

                    carry = lax.cond(cmax >= tau, ins, lambda c: c, carry)
                return carry

            return lax.cond(jnp.max(bvec) + emax_c >= tau0, do_row,
                            lambda c: c, carry)

        fbufs = _unflatten(lax.fori_loop(0, M, frow, _init_bufs(NB_K)))

        for t in range(NB_K):
            fk, fv = fbufs[t]
            oiv[pl.ds(t * L, L)] = lax.shift_right_logical(fv, 13)
            ojv[pl.ds(t * L, L)] = fv & (S - 1)
            ovv[pl.ds(t * L, L)] = jnp.exp(fk) / denom
        pltpu.sync_copy(oiv, oi_hbm)
        pltpu.sync_copy(ojv, oj_hbm)
        pltpu.sync_copy(ovv, ov_hbm)


@jax.jit
def _sc_select(b, e):
    mesh = plsc.VectorSubcoreMesh(core_axis_name="c", subcore_axis_name="s")
    lists = functools.partial(
        pl.kernel,
        mesh=mesh,
        compiler_params=pltpu.CompilerParams(needs_layout_passes=False),
        out_type=[
            jax.ShapeDtypeStruct((NW * M,), jnp.float32),
            jax.ShapeDtypeStruct((NW * M,), jnp.int32),
            jax.ShapeDtypeStruct((NW * M,), jnp.float32),
            jax.ShapeDtypeStruct((NW * M,), jnp.int32),
            jax.ShapeDtypeStruct((NW * L,), jnp.float32),
            jax.ShapeDtypeStruct((NW * L,), jnp.float32),
        ],
        scratch_types=[
            pltpu.VMEM((S,), jnp.float32),
            pltpu.VMEM((S,), jnp.float32),
            pltpu.VMEM((M,), jnp.float32),
            pltpu.VMEM((M,), jnp.int32),
            pltpu.VMEM((L,), jnp.float32),
        ],
    )(_lists_body)
    ck, ci, dk, di, sb, se = lists(b, e)

    final = functools.partial(
        pl.kernel,
        mesh=mesh,
        compiler_params=pltpu.CompilerParams(needs_layout_passes=False),
        out_type=[
            jax.ShapeDtypeStruct((TOPK,), jnp.int32),
            jax.ShapeDtypeStruct((TOPK,), jnp.int32),
            jax.ShapeDtypeStruct((TOPK,), jnp.float32),
        ],
        scratch_types=[
            pltpu.VMEM((S,), jnp.float32),
            pltpu.VMEM((S,), jnp.float32),
            pltpu.VMEM((NW * M,), jnp.float32),
            pltpu.VMEM((NW * M,), jnp.int32),
            pltpu.VMEM((M,), jnp.float32),
            pltpu.VMEM((M,), jnp.int32),
            pltpu.VMEM((M,), jnp.float32),
            pltpu.VMEM((M,), jnp.int32),
            pltpu.VMEM((NW * L,), jnp.float32),
            pltpu.VMEM((TOPK,), jnp.int32),
            pltpu.VMEM((TOPK,), jnp.int32),
            pltpu.VMEM((TOPK,), jnp.float32),
        ],
    )(_final_body)
    return final(b, e, ck, ci, dk, di, sb, se)


def _tc_matvec(G, W):
    def body(g_ref, w_ref, o_ref):
        o_ref[...] = jnp.dot(g_ref[...], w_ref[...],
                             preferred_element_type=jnp.float32)

    return pl.pallas_call(
        body,
        out_shape=jax.ShapeDtypeStruct((S, 2), jnp.float32),
    )(G, W)


def kernel(G_p, Wb, We, k):
    del k
    be = _tc_matvec(G_p, jnp.concatenate([Wb, We], axis=1))
    b = be[:, 0]
    e = be[:, 1]
    oi, oj, vals = _sc_select(b, e)
    return (jnp.concatenate([oi[:, None], oj[:, None]], axis=1), vals)
